# baseline (speedup 1.0000x reference)
import functools

import jax
import jax.numpy as jnp
from jax import lax
from jax.experimental import pallas as pl
from jax.experimental.pallas import tpu as pltpu
from jax.experimental.pallas import tpu_sc as plsc

N = 10000
E = 320000
D = 128

NC = 2
NS = 16
NW = NC * NS
CH = 64
ITERS = 160
EPW = CH * ITERS
E_PAD = NW * EPW
N_ACC = 10128
RPS = 624
TAIL0 = NS * RPS
TAIL = N - TAIL0

ROW_BLK = 1000


def _mm_body(h_ref, w_ref, norm_ref, o_ref):
    o_ref[...] = (
        jnp.dot(h_ref[...], w_ref[...], preferred_element_type=jnp.float32)
        * norm_ref[...]
    )


def _fin_body(p0_ref, p1_ref, norm_ref, bias_ref, o_ref):
    o_ref[...] = (p0_ref[...] + p1_ref[...]) * norm_ref[...] + bias_ref[...]


@functools.partial(
    pl.kernel,
    mesh=plsc.VectorSubcoreMesh(core_axis_name="c", subcore_axis_name="s"),
    out_type=jax.ShapeDtypeStruct((NC, N, D), jnp.float32),
    scratch_types=[
        pltpu.VMEM((CH,), jnp.int32),
        pltpu.VMEM((CH,), jnp.int32),
        pltpu.VMEM((CH, D), jnp.float32),
        pltpu.VMEM_SHARED((N_ACC, D), jnp.float32),
        pltpu.SemaphoreType.DMA,
    ],
)
def _sc_edge(xs_hbm, src_hbm, dst_hbm, zeros_hbm, out_hbm,
             src_v, dst_v, rows_v, acc_sh, sem):
    c = lax.axis_index("c")
    s = lax.axis_index("s")
    r0 = s * RPS
    pltpu.sync_copy(zeros_hbm.at[pl.ds(r0, RPS)], acc_sh.at[pl.ds(r0, RPS)])

    @pl.when(s == 0)
    def _init_tail():
        pltpu.sync_copy(zeros_hbm.at[pl.ds(TAIL0, TAIL)],
                        acc_sh.at[pl.ds(TAIL0, TAIL)])

    plsc.subcore_barrier()

    base = (c * NS + s) * EPW

    def body(i, carry):
        off = base + i * CH
        pltpu.sync_copy(src_hbm.at[pl.ds(off, CH)], src_v)
        pltpu.sync_copy(dst_hbm.at[pl.ds(off, CH)], dst_v)
        pltpu.async_copy(xs_hbm.at[src_v], rows_v, sem).wait()
        pltpu.sync_copy(rows_v, acc_sh.at[dst_v], add=True)
        return carry

    lax.fori_loop(0, ITERS, body, 0)
    plsc.subcore_barrier()
    pltpu.sync_copy(acc_sh.at[pl.ds(r0, RPS)], out_hbm.at[c, pl.ds(r0, RPS)])

    @pl.when(s == 0)
    def _out_tail():
        pltpu.sync_copy(acc_sh.at[pl.ds(TAIL0, TAIL)],
                        out_hbm.at[c, pl.ds(TAIL0, TAIL)])


def kernel(h, edge_index, W, bias, norm):
    pad = E_PAD - E
    src = jnp.concatenate([edge_index[0], jnp.zeros((pad,), jnp.int32)])
    dst = jnp.concatenate(
        [edge_index[1], N + (jnp.arange(pad, dtype=jnp.int32) % CH)])
    normc = norm[:, None]

    xs = pl.pallas_call(
        _mm_body,
        grid=(N // ROW_BLK,),
        in_specs=[
            pl.BlockSpec((ROW_BLK, D), lambda i: (i, 0)),
            pl.BlockSpec((D, D), lambda i: (0, 0)),
            pl.BlockSpec((ROW_BLK, 1), lambda i: (i, 0)),
        ],
        out_specs=pl.BlockSpec((ROW_BLK, D), lambda i: (i, 0)),
        out_shape=jax.ShapeDtypeStruct((N, D), jnp.float32),
    )(h, W, normc)

    zeros = jnp.zeros((N, D), jnp.float32)
    partial = _sc_edge(xs, src, dst, zeros)

    out = pl.pallas_call(
        _fin_body,
        grid=(N // ROW_BLK,),
        in_specs=[
            pl.BlockSpec((ROW_BLK, D), lambda i: (i, 0)),
            pl.BlockSpec((ROW_BLK, D), lambda i: (i, 0)),
            pl.BlockSpec((ROW_BLK, 1), lambda i: (i, 0)),
            pl.BlockSpec((1, D), lambda i: (0, 0)),
        ],
        out_specs=pl.BlockSpec((ROW_BLK, D), lambda i: (i, 0)),
        out_shape=jax.ShapeDtypeStruct((N, D), jnp.float32),
    )(partial[0], partial[1], normc, bias.reshape(1, D))
    return out


# speedup vs baseline: 1.7275x; 1.7275x over previous
import functools

import jax
import jax.numpy as jnp
from jax import lax
from jax.experimental import pallas as pl
from jax.experimental.pallas import tpu as pltpu
from jax.experimental.pallas import tpu_sc as plsc

N = 10000
E = 320000
D = 128

NC = 2
NS = 16
NW = NC * NS
CH = 64
ITERS = 160
EPW = CH * ITERS
E_PAD = NW * EPW
N_ACC = 10128
RPS = 624
TAIL0 = NS * RPS
TAIL = N - TAIL0

ROW_BLK = 1000


def _mm_body(h_ref, w_ref, norm_ref, o_ref):
    o_ref[...] = (
        jnp.dot(h_ref[...], w_ref[...], preferred_element_type=jnp.float32)
        * norm_ref[...]
    )


def _fin_body(p0_ref, p1_ref, norm_ref, bias_ref, o_ref):
    o_ref[...] = (p0_ref[...] + p1_ref[...]) * norm_ref[...] + bias_ref[...]


@functools.partial(
    pl.kernel,
    mesh=plsc.VectorSubcoreMesh(core_axis_name="c", subcore_axis_name="s"),
    out_type=jax.ShapeDtypeStruct((NC, N, D), jnp.float32),
    scratch_types=[
        pltpu.VMEM((CH,), jnp.int32),
        pltpu.VMEM((CH,), jnp.int32),
        pltpu.VMEM((CH, D), jnp.float32),
        pltpu.VMEM_SHARED((N_ACC, D), jnp.float32),
        pltpu.SemaphoreType.DMA,
    ],
)
def _sc_edge(xs_hbm, src_hbm, dst_hbm, zeros_hbm, out_hbm,
             src_v, dst_v, rows_v, acc_sh, sem):
    c = lax.axis_index("c")
    s = lax.axis_index("s")
    r0 = s * RPS
    pltpu.sync_copy(zeros_hbm.at[pl.ds(r0, RPS)], acc_sh.at[pl.ds(r0, RPS)])

    @pl.when(s == 0)
    def _init_tail():
        pltpu.sync_copy(zeros_hbm.at[pl.ds(TAIL0, TAIL)],
                        acc_sh.at[pl.ds(TAIL0, TAIL)])

    plsc.subcore_barrier()

    base = (c * NS + s) * EPW

    def body(i, carry):
        off = base + i * CH
        pltpu.sync_copy(src_hbm.at[pl.ds(off, CH)], src_v)
        pltpu.sync_copy(dst_hbm.at[pl.ds(off, CH)], dst_v)
        pltpu.async_copy(xs_hbm.at[src_v], rows_v, sem).wait()
        pltpu.sync_copy(rows_v, acc_sh.at[dst_v], add=True)
        return carry

    lax.fori_loop(0, ITERS, body, 0)
    plsc.subcore_barrier()
    pltpu.sync_copy(acc_sh.at[pl.ds(r0, RPS)], out_hbm.at[c, pl.ds(r0, RPS)])

    @pl.when(s == 0)
    def _out_tail():
        pltpu.sync_copy(acc_sh.at[pl.ds(TAIL0, TAIL)],
                        out_hbm.at[c, pl.ds(TAIL0, TAIL)])


def kernel(h, edge_index, W, bias, norm):
    pad = E_PAD - E
    src = jnp.concatenate(
        [edge_index[0], jnp.arange(pad, dtype=jnp.int32) % 1024])
    dst = jnp.concatenate(
        [edge_index[1], N + (jnp.arange(pad, dtype=jnp.int32) % CH)])
    normc = norm[:, None]

    xs = pl.pallas_call(
        _mm_body,
        grid=(N // ROW_BLK,),
        in_specs=[
            pl.BlockSpec((ROW_BLK, D), lambda i: (i, 0)),
            pl.BlockSpec((D, D), lambda i: (0, 0)),
            pl.BlockSpec((ROW_BLK, 1), lambda i: (i, 0)),
        ],
        out_specs=pl.BlockSpec((ROW_BLK, D), lambda i: (i, 0)),
        out_shape=jax.ShapeDtypeStruct((N, D), jnp.float32),
    )(h, W, normc)

    zeros = jnp.zeros((N, D), jnp.float32)
    partial = _sc_edge(xs, src, dst, zeros)

    out = pl.pallas_call(
        _fin_body,
        grid=(N // ROW_BLK,),
        in_specs=[
            pl.BlockSpec((ROW_BLK, D), lambda i: (i, 0)),
            pl.BlockSpec((ROW_BLK, D), lambda i: (i, 0)),
            pl.BlockSpec((ROW_BLK, 1), lambda i: (i, 0)),
            pl.BlockSpec((1, D), lambda i: (0, 0)),
        ],
        out_specs=pl.BlockSpec((ROW_BLK, D), lambda i: (i, 0)),
        out_shape=jax.ShapeDtypeStruct((N, D), jnp.float32),
    )(partial[0], partial[1], normc, bias.reshape(1, D))
    return out


# speedup vs baseline: 2.3532x; 1.3622x over previous
import functools

import jax
import jax.numpy as jnp
from jax import lax
from jax.experimental import pallas as pl
from jax.experimental.pallas import tpu as pltpu
from jax.experimental.pallas import tpu_sc as plsc

N = 10000
E = 320000
D = 128

NC = 2
NS = 16
NW = NC * NS
CH = 128
ITERS = 80
EPW = CH * ITERS
E_PAD = NW * EPW
N_ACC = 10128
RPS = 624
TAIL0 = NS * RPS
TAIL = N - TAIL0

ROW_BLK = 1000


def _mm_body(h_ref, w_ref, norm_ref, o_ref):
    o_ref[...] = (
        jnp.dot(h_ref[...], w_ref[...], preferred_element_type=jnp.float32)
        * norm_ref[...]
    )


def _fin_body(p0_ref, p1_ref, norm_ref, bias_ref, o_ref):
    o_ref[...] = (p0_ref[...] + p1_ref[...]) * norm_ref[...] + bias_ref[...]


@functools.partial(
    pl.kernel,
    mesh=plsc.VectorSubcoreMesh(core_axis_name="c", subcore_axis_name="s"),
    out_type=jax.ShapeDtypeStruct((NC, N, D), jnp.float32),
    scratch_types=[
        pltpu.VMEM((CH,), jnp.int32),
        pltpu.VMEM((CH,), jnp.int32),
        pltpu.VMEM((CH, D), jnp.float32),
        pltpu.VMEM_SHARED((N_ACC, D), jnp.float32),
        pltpu.SemaphoreType.DMA,
    ],
)
def _sc_edge(xs_hbm, src_hbm, dst_hbm, zeros_hbm, out_hbm,
             src_v, dst_v, rows_v, acc_sh, sem):
    c = lax.axis_index("c")
    s = lax.axis_index("s")
    r0 = s * RPS
    pltpu.sync_copy(zeros_hbm.at[pl.ds(r0, RPS)], acc_sh.at[pl.ds(r0, RPS)])

    @pl.when(s == 0)
    def _init_tail():
        pltpu.sync_copy(zeros_hbm.at[pl.ds(TAIL0, TAIL)],
                        acc_sh.at[pl.ds(TAIL0, TAIL)])

    plsc.subcore_barrier()

    base = (c * NS + s) * EPW

    def body(i, carry):
        off = base + i * CH
        pltpu.sync_copy(src_hbm.at[pl.ds(off, CH)], src_v)
        pltpu.sync_copy(dst_hbm.at[pl.ds(off, CH)], dst_v)
        pltpu.async_copy(xs_hbm.at[src_v], rows_v, sem).wait()
        pltpu.sync_copy(rows_v, acc_sh.at[dst_v], add=True)
        return carry

    lax.fori_loop(0, ITERS, body, 0)
    plsc.subcore_barrier()
    pltpu.sync_copy(acc_sh.at[pl.ds(r0, RPS)], out_hbm.at[c, pl.ds(r0, RPS)])

    @pl.when(s == 0)
    def _out_tail():
        pltpu.sync_copy(acc_sh.at[pl.ds(TAIL0, TAIL)],
                        out_hbm.at[c, pl.ds(TAIL0, TAIL)])


def kernel(h, edge_index, W, bias, norm):
    pad = E_PAD - E
    src = jnp.concatenate(
        [edge_index[0], jnp.arange(pad, dtype=jnp.int32) % 1024])
    dst = jnp.concatenate(
        [edge_index[1], N + (jnp.arange(pad, dtype=jnp.int32) % CH)])
    normc = norm[:, None]

    xs = pl.pallas_call(
        _mm_body,
        grid=(N // ROW_BLK,),
        in_specs=[
            pl.BlockSpec((ROW_BLK, D), lambda i: (i, 0)),
            pl.BlockSpec((D, D), lambda i: (0, 0)),
            pl.BlockSpec((ROW_BLK, 1), lambda i: (i, 0)),
        ],
        out_specs=pl.BlockSpec((ROW_BLK, D), lambda i: (i, 0)),
        out_shape=jax.ShapeDtypeStruct((N, D), jnp.float32),
    )(h, W, normc)

    zeros = jnp.zeros((N, D), jnp.float32)
    partial = _sc_edge(xs, src, dst, zeros)

    out = pl.pallas_call(
        _fin_body,
        grid=(N // ROW_BLK,),
        in_specs=[
            pl.BlockSpec((ROW_BLK, D), lambda i: (i, 0)),
            pl.BlockSpec((ROW_BLK, D), lambda i: (i, 0)),
            pl.BlockSpec((ROW_BLK, 1), lambda i: (i, 0)),
            pl.BlockSpec((1, D), lambda i: (0, 0)),
        ],
        out_specs=pl.BlockSpec((ROW_BLK, D), lambda i: (i, 0)),
        out_shape=jax.ShapeDtypeStruct((N, D), jnp.float32),
    )(partial[0], partial[1], normc, bias.reshape(1, D))
    return out


# speedup vs baseline: 3.8933x; 1.6545x over previous
import functools

import jax
import jax.numpy as jnp
from jax import lax
from jax.experimental import pallas as pl
from jax.experimental.pallas import tpu as pltpu
from jax.experimental.pallas import tpu_sc as plsc

N = 10000
E = 320000
D = 128

NC = 2
NS = 16
NW = NC * NS
CH = 128
ITERS = 80
EPW = CH * ITERS
E_MAIN = NW * EPW
E_PAD = E_MAIN + CH
N_ACC = 10128
RPS = 624
TAIL0 = NS * RPS
TAIL = N - TAIL0

ROW_BLK = 1000


def _mm_body(h_ref, w_ref, norm_ref, o_ref):
    o_ref[...] = (
        jnp.dot(h_ref[...], w_ref[...], preferred_element_type=jnp.float32)
        * norm_ref[...]
    )


def _fin_body(p0_ref, p1_ref, norm_ref, bias_ref, o_ref):
    o_ref[...] = (p0_ref[...] + p1_ref[...]) * norm_ref[...] + bias_ref[...]


@functools.partial(
    pl.kernel,
    mesh=plsc.VectorSubcoreMesh(core_axis_name="c", subcore_axis_name="s"),
    out_type=jax.ShapeDtypeStruct((NC, N, D), jnp.float32),
    scratch_types=[
        pltpu.VMEM((CH,), jnp.int32),
        pltpu.VMEM((CH,), jnp.int32),
        pltpu.VMEM((CH,), jnp.int32),
        pltpu.VMEM((CH,), jnp.int32),
        pltpu.VMEM((CH, D), jnp.float32),
        pltpu.VMEM((CH, D), jnp.float32),
        pltpu.VMEM_SHARED((N_ACC, D), jnp.float32),
        pltpu.SemaphoreType.DMA,
        pltpu.SemaphoreType.DMA,
        pltpu.SemaphoreType.DMA,
        pltpu.SemaphoreType.DMA,
    ],
)
def _sc_edge(xs_hbm, src_hbm, dst_hbm, zeros_hbm, out_hbm,
             sbuf0, sbuf1, dbuf0, dbuf1, rows0, rows1, acc_sh,
             semi0, semi1, semg0, semg1):
    c = lax.axis_index("c")
    s = lax.axis_index("s")
    sbuf = [sbuf0, sbuf1]
    dbuf = [dbuf0, dbuf1]
    rows = [rows0, rows1]
    semi = [semi0, semi1]
    semg = [semg0, semg1]
    r0 = s * RPS
    pltpu.sync_copy(zeros_hbm.at[pl.ds(r0, RPS)], acc_sh.at[pl.ds(r0, RPS)])

    @pl.when(s == 0)
    def _init_tail():
        pltpu.sync_copy(zeros_hbm.at[pl.ds(TAIL0, TAIL)],
                        acc_sh.at[pl.ds(TAIL0, TAIL)])

    plsc.subcore_barrier()

    base = (c * NS + s) * EPW

    def idx_start(b, off):
        pltpu.make_async_copy(
            src_hbm.at[pl.ds(off, CH)], sbuf[b], semi[b]).start()
        pltpu.make_async_copy(
            dst_hbm.at[pl.ds(off, CH)], dbuf[b], semi[b]).start()

    def idx_wait(b):
        pltpu.make_async_copy(
            src_hbm.at[pl.ds(0, CH)], sbuf[b], semi[b]).wait()
        pltpu.make_async_copy(
            dst_hbm.at[pl.ds(0, CH)], dbuf[b], semi[b]).wait()

    def g_start(b):
        pltpu.make_async_copy(
            xs_hbm.at[sbuf[b]], rows[b], semg[b]).start()

    def g_wait(b):
        pltpu.make_async_copy(
            xs_hbm.at[sbuf[b]], rows[b], semg[b]).wait()

    idx_start(0, base)
    idx_wait(0)
    g_start(0)
    idx_start(1, base + CH)

    def body(k, b):
        nb = b ^ 1
        idx_wait(b)
        g_start(b)
        g_wait(nb)
        pltpu.sync_copy(rows[nb], acc_sh.at[dbuf[nb]], add=True)
        idx_start(nb, base + (k + 1) * CH)

    body(1, 1)

    def loop_body(j, carry):
        body(2 * j, 0)
        body(2 * j + 1, 1)
        return carry

    lax.fori_loop(1, ITERS // 2, loop_body, 0)

    g_wait(1)
    pltpu.sync_copy(rows[1], acc_sh.at[dbuf[1]], add=True)
    idx_wait(0)
    plsc.subcore_barrier()
    pltpu.sync_copy(acc_sh.at[pl.ds(r0, RPS)], out_hbm.at[c, pl.ds(r0, RPS)])

    @pl.when(s == 0)
    def _out_tail():
        pltpu.sync_copy(acc_sh.at[pl.ds(TAIL0, TAIL)],
                        out_hbm.at[c, pl.ds(TAIL0, TAIL)])


def kernel(h, edge_index, W, bias, norm):
    pad = E_PAD - E
    src = jnp.concatenate(
        [edge_index[0], jnp.arange(pad, dtype=jnp.int32) % 1024])
    dst = jnp.concatenate(
        [edge_index[1], N + (jnp.arange(pad, dtype=jnp.int32) % CH)])
    normc = norm[:, None]

    xs = pl.pallas_call(
        _mm_body,
        grid=(N // ROW_BLK,),
        in_specs=[
            pl.BlockSpec((ROW_BLK, D), lambda i: (i, 0)),
            pl.BlockSpec((D, D), lambda i: (0, 0)),
            pl.BlockSpec((ROW_BLK, 1), lambda i: (i, 0)),
        ],
        out_specs=pl.BlockSpec((ROW_BLK, D), lambda i: (i, 0)),
        out_shape=jax.ShapeDtypeStruct((N, D), jnp.float32),
    )(h, W, normc)

    zeros = jnp.zeros((N, D), jnp.float32)
    partial = _sc_edge(xs, src, dst, zeros)

    out = pl.pallas_call(
        _fin_body,
        grid=(N // ROW_BLK,),
        in_specs=[
            pl.BlockSpec((ROW_BLK, D), lambda i: (i, 0)),
            pl.BlockSpec((ROW_BLK, D), lambda i: (i, 0)),
            pl.BlockSpec((ROW_BLK, 1), lambda i: (i, 0)),
            pl.BlockSpec((1, D), lambda i: (0, 0)),
        ],
        out_specs=pl.BlockSpec((ROW_BLK, D), lambda i: (i, 0)),
        out_shape=jax.ShapeDtypeStruct((N, D), jnp.float32),
    )(partial[0], partial[1], normc, bias.reshape(1, D))
    return out


# speedup vs baseline: 4.2146x; 1.0825x over previous
import functools

import jax
import jax.numpy as jnp
from jax import lax
from jax.experimental import pallas as pl
from jax.experimental.pallas import tpu as pltpu
from jax.experimental.pallas import tpu_sc as plsc

N = 10000
E = 320000
D = 128

NC = 2
NS = 16
NW = NC * NS
CH = 128
ITERS = 80
EPW = CH * ITERS
E_MAIN = NW * EPW
E_PAD = E_MAIN + CH
N_ACC = 10128
RPS = 624
TAIL0 = NS * RPS
TAIL = N - TAIL0

ROW_BLK = 1000


def _mm_body(h_ref, w_ref, norm_ref, o_ref):
    o_ref[...] = (
        jnp.dot(h_ref[...], w_ref[...], preferred_element_type=jnp.float32)
        * norm_ref[...]
    )


def _fin_body(p0_ref, p1_ref, norm_ref, bias_ref, o_ref):
    o_ref[...] = (p0_ref[...] + p1_ref[...]) * norm_ref[...] + bias_ref[...]


@functools.partial(
    pl.kernel,
    mesh=plsc.VectorSubcoreMesh(core_axis_name="c", subcore_axis_name="s"),
    out_type=jax.ShapeDtypeStruct((NC, N, D), jnp.float32),
    scratch_types=[
        pltpu.VMEM((CH,), jnp.int32),
        pltpu.VMEM((CH,), jnp.int32),
        pltpu.VMEM((CH,), jnp.int32),
        pltpu.VMEM((CH,), jnp.int32),
        pltpu.VMEM((CH,), jnp.int32),
        pltpu.VMEM((CH,), jnp.int32),
        pltpu.VMEM((CH, D), jnp.float32),
        pltpu.VMEM((CH, D), jnp.float32),
        pltpu.VMEM_SHARED((N_ACC, D), jnp.float32),
        pltpu.SemaphoreType.DMA,
        pltpu.SemaphoreType.DMA,
        pltpu.SemaphoreType.DMA,
        pltpu.SemaphoreType.DMA,
        pltpu.SemaphoreType.DMA,
        pltpu.SemaphoreType.DMA,
    ],
)
def _sc_edge(xs_hbm, src_hbm, dst_hbm, zeros_hbm, out_hbm,
             sbuf0, sbuf1, dbuf0, dbuf1, sidx0, sidx1, rows0, rows1, acc_sh,
             semi0, semi1, semg0, semg1, sems0, sems1):
    c = lax.axis_index("c")
    s = lax.axis_index("s")
    sbuf = [sbuf0, sbuf1]
    dbuf = [dbuf0, dbuf1]
    sidx = [sidx0, sidx1]
    rows = [rows0, rows1]
    semi = [semi0, semi1]
    semg = [semg0, semg1]
    sems = [sems0, sems1]
    r0 = s * RPS
    pltpu.sync_copy(zeros_hbm.at[pl.ds(r0, RPS)], acc_sh.at[pl.ds(r0, RPS)])

    @pl.when(s == 0)
    def _init_tail():
        pltpu.sync_copy(zeros_hbm.at[pl.ds(TAIL0, TAIL)],
                        acc_sh.at[pl.ds(TAIL0, TAIL)])

    plsc.subcore_barrier()

    base = (c * NS + s) * EPW

    def idx_start(b, off):
        pltpu.make_async_copy(
            src_hbm.at[pl.ds(off, CH)], sbuf[b], semi[b]).start()
        pltpu.make_async_copy(
            dst_hbm.at[pl.ds(off, CH)], dbuf[b], semi[b]).start()

    def idx_wait(b):
        pltpu.make_async_copy(
            src_hbm.at[pl.ds(0, CH)], sbuf[b], semi[b]).wait()
        pltpu.make_async_copy(
            dst_hbm.at[pl.ds(0, CH)], dbuf[b], semi[b]).wait()

    def g_start(b):
        pltpu.make_async_copy(
            xs_hbm.at[sbuf[b]], rows[b], semg[b]).start()

    def g_wait(b):
        pltpu.make_async_copy(
            xs_hbm.at[sbuf[b]], rows[b], semg[b]).wait()

    def snap_sidx(b):
        for tt in range(CH // 16):
            sidx[b][pl.ds(tt * 16, 16)] = dbuf[b][pl.ds(tt * 16, 16)]

    def s_start(b):
        pltpu.make_async_copy(
            rows[b], acc_sh.at[sidx[b]], sems[b]).start(add=True)

    def s_wait(b):
        pltpu.make_async_copy(
            rows[b], acc_sh.at[sidx[b]], sems[b]).wait()

    idx_start(0, base)
    idx_wait(0)
    g_start(0)
    idx_start(1, base + CH)
    idx_wait(1)
    g_start(1)
    g_wait(0)
    snap_sidx(0)
    s_start(0)
    idx_start(0, base + 2 * CH)

    def body(k, b):
        nb = b ^ 1
        s_wait(b)
        idx_wait(b)
        g_start(b)
        g_wait(nb)
        snap_sidx(nb)
        s_start(nb)
        idx_start(nb, base + (k + 1) * CH)

    def loop_body(j, carry):
        body(2 * j, 0)
        body(2 * j + 1, 1)
        return carry

    lax.fori_loop(1, ITERS // 2, loop_body, 0)

    g_wait(1)
    snap_sidx(1)
    s_start(1)
    s_wait(0)
    s_wait(1)
    idx_wait(0)
    plsc.subcore_barrier()
    pltpu.sync_copy(acc_sh.at[pl.ds(r0, RPS)], out_hbm.at[c, pl.ds(r0, RPS)])

    @pl.when(s == 0)
    def _out_tail():
        pltpu.sync_copy(acc_sh.at[pl.ds(TAIL0, TAIL)],
                        out_hbm.at[c, pl.ds(TAIL0, TAIL)])


def kernel(h, edge_index, W, bias, norm):
    pad = E_PAD - E
    src = jnp.concatenate(
        [edge_index[0], jnp.arange(pad, dtype=jnp.int32) % 1024])
    dst = jnp.concatenate(
        [edge_index[1], N + (jnp.arange(pad, dtype=jnp.int32) % CH)])
    normc = norm[:, None]

    xs = pl.pallas_call(
        _mm_body,
        grid=(N // ROW_BLK,),
        in_specs=[
            pl.BlockSpec((ROW_BLK, D), lambda i: (i, 0)),
            pl.BlockSpec((D, D), lambda i: (0, 0)),
            pl.BlockSpec((ROW_BLK, 1), lambda i: (i, 0)),
        ],
        out_specs=pl.BlockSpec((ROW_BLK, D), lambda i: (i, 0)),
        out_shape=jax.ShapeDtypeStruct((N, D), jnp.float32),
    )(h, W, normc)

    zeros = jnp.zeros((N, D), jnp.float32)
    partial = _sc_edge(xs, src, dst, zeros)

    out = pl.pallas_call(
        _fin_body,
        grid=(N // ROW_BLK,),
        in_specs=[
            pl.BlockSpec((ROW_BLK, D), lambda i: (i, 0)),
            pl.BlockSpec((ROW_BLK, D), lambda i: (i, 0)),
            pl.BlockSpec((ROW_BLK, 1), lambda i: (i, 0)),
            pl.BlockSpec((1, D), lambda i: (0, 0)),
        ],
        out_specs=pl.BlockSpec((ROW_BLK, D), lambda i: (i, 0)),
        out_shape=jax.ShapeDtypeStruct((N, D), jnp.float32),
    )(partial[0], partial[1], normc, bias.reshape(1, D))
    return out


# speedup vs baseline: 4.2571x; 1.0101x over previous
import functools

import jax
import jax.numpy as jnp
from jax import lax
from jax.experimental import pallas as pl
from jax.experimental.pallas import tpu as pltpu
from jax.experimental.pallas import tpu_sc as plsc

N = 10000
E = 320000
D = 128

NC = 2
NS = 16
NW = NC * NS
CH = 128
EPW = E // NW
ITERS = EPW // CH
TCH = EPW - ITERS * CH
RPS = 624
TAIL0 = NS * RPS
TAIL = N - TAIL0

ROW_BLK = 1000


def _mm_body(h_ref, w_ref, norm_ref, o_ref):
    o_ref[...] = (
        jnp.dot(h_ref[...], w_ref[...], preferred_element_type=jnp.float32)
        * norm_ref[...]
    )


def _fin_body(p0_ref, p1_ref, norm_ref, bias_ref, o_ref):
    o_ref[...] = (p0_ref[...] + p1_ref[...]) * norm_ref[...] + bias_ref[...]


@functools.partial(
    pl.kernel,
    mesh=plsc.VectorSubcoreMesh(core_axis_name="c", subcore_axis_name="s"),
    out_type=jax.ShapeDtypeStruct((NC, N, D), jnp.float32),
    scratch_types=[
        pltpu.VMEM((CH,), jnp.int32),
        pltpu.VMEM((CH,), jnp.int32),
        pltpu.VMEM((CH,), jnp.int32),
        pltpu.VMEM((CH,), jnp.int32),
        pltpu.VMEM((CH,), jnp.int32),
        pltpu.VMEM((CH,), jnp.int32),
        pltpu.VMEM((CH, D), jnp.float32),
        pltpu.VMEM((CH, D), jnp.float32),
        pltpu.VMEM((TCH,), jnp.int32),
        pltpu.VMEM((TCH,), jnp.int32),
        pltpu.VMEM((TCH, D), jnp.float32),
        pltpu.VMEM_SHARED((N, D), jnp.float32),
        pltpu.SemaphoreType.DMA,
        pltpu.SemaphoreType.DMA,
        pltpu.SemaphoreType.DMA,
        pltpu.SemaphoreType.DMA,
        pltpu.SemaphoreType.DMA,
        pltpu.SemaphoreType.DMA,
    ],
)
def _sc_edge(xs_hbm, src_hbm, dst_hbm, zeros_hbm, out_hbm,
             sbuf0, sbuf1, dbuf0, dbuf1, sidx0, sidx1, rows0, rows1,
             tsrc, tdst, trows, acc_sh,
             semi0, semi1, semg0, semg1, sems0, sems1):
    c = lax.axis_index("c")
    s = lax.axis_index("s")
    sbuf = [sbuf0, sbuf1]
    dbuf = [dbuf0, dbuf1]
    sidx = [sidx0, sidx1]
    rows = [rows0, rows1]
    semi = [semi0, semi1]
    semg = [semg0, semg1]
    sems = [sems0, sems1]
    r0 = s * RPS
    pltpu.sync_copy(zeros_hbm.at[pl.ds(r0, RPS)], acc_sh.at[pl.ds(r0, RPS)])

    @pl.when(s == 0)
    def _init_tail():
        pltpu.sync_copy(zeros_hbm.at[pl.ds(TAIL0, TAIL)],
                        acc_sh.at[pl.ds(TAIL0, TAIL)])

    plsc.subcore_barrier()

    base = (c * NS + s) * EPW

    def idx_start(b, off):
        off = jnp.minimum(off, E - CH)
        pltpu.make_async_copy(
            src_hbm.at[pl.ds(off, CH)], sbuf[b], semi[b]).start()
        pltpu.make_async_copy(
            dst_hbm.at[pl.ds(off, CH)], dbuf[b], semi[b]).start()

    def idx_wait(b):
        pltpu.make_async_copy(
            src_hbm.at[pl.ds(0, CH)], sbuf[b], semi[b]).wait()
        pltpu.make_async_copy(
            dst_hbm.at[pl.ds(0, CH)], dbuf[b], semi[b]).wait()

    def g_start(b):
        pltpu.make_async_copy(
            xs_hbm.at[sbuf[b]], rows[b], semg[b]).start()

    def g_wait(b):
        pltpu.make_async_copy(
            xs_hbm.at[sbuf[b]], rows[b], semg[b]).wait()

    def snap_sidx(b):
        for tt in range(CH // 16):
            sidx[b][pl.ds(tt * 16, 16)] = dbuf[b][pl.ds(tt * 16, 16)]

    def s_start(b):
        pltpu.make_async_copy(
            rows[b], acc_sh.at[sidx[b]], sems[b]).start(add=True)

    def s_wait(b):
        pltpu.make_async_copy(
            rows[b], acc_sh.at[sidx[b]], sems[b]).wait()

    idx_start(0, base)
    idx_wait(0)
    g_start(0)
    idx_start(1, base + CH)
    idx_wait(1)
    g_start(1)
    g_wait(0)
    snap_sidx(0)
    s_start(0)
    idx_start(0, base + 2 * CH)

    def body(k, b):
        nb = b ^ 1
        s_wait(b)
        idx_wait(b)
        g_start(b)
        g_wait(nb)
        snap_sidx(nb)
        s_start(nb)
        idx_start(nb, base + (k + 1) * CH)

    def loop_body(j, carry):
        body(2 * j, 0)
        body(2 * j + 1, 1)
        return carry

    lax.fori_loop(1, ITERS // 2, loop_body, 0)

    g_wait(1)
    snap_sidx(1)
    s_start(1)
    s_wait(0)
    s_wait(1)
    idx_wait(0)

    toff = base + ITERS * CH
    pltpu.sync_copy(src_hbm.at[pl.ds(toff, TCH)], tsrc)
    pltpu.sync_copy(dst_hbm.at[pl.ds(toff, TCH)], tdst)
    pltpu.async_copy(xs_hbm.at[tsrc], trows, semg0).wait()
    pltpu.sync_copy(trows, acc_sh.at[tdst], add=True)

    plsc.subcore_barrier()
    pltpu.sync_copy(acc_sh.at[pl.ds(r0, RPS)], out_hbm.at[c, pl.ds(r0, RPS)])

    @pl.when(s == 0)
    def _out_tail():
        pltpu.sync_copy(acc_sh.at[pl.ds(TAIL0, TAIL)],
                        out_hbm.at[c, pl.ds(TAIL0, TAIL)])


def kernel(h, edge_index, W, bias, norm):
    src = edge_index[0]
    dst = edge_index[1]
    normc = norm[:, None]

    xs = pl.pallas_call(
        _mm_body,
        grid=(N // ROW_BLK,),
        in_specs=[
            pl.BlockSpec((ROW_BLK, D), lambda i: (i, 0)),
            pl.BlockSpec((D, D), lambda i: (0, 0)),
            pl.BlockSpec((ROW_BLK, 1), lambda i: (i, 0)),
        ],
        out_specs=pl.BlockSpec((ROW_BLK, D), lambda i: (i, 0)),
        out_shape=jax.ShapeDtypeStruct((N, D), jnp.float32),
    )(h, W, normc)

    zeros = jnp.zeros((N, D), jnp.float32)
    partial = _sc_edge(xs, src, dst, zeros)

    out = pl.pallas_call(
        _fin_body,
        grid=(N // ROW_BLK,),
        in_specs=[
            pl.BlockSpec((ROW_BLK, D), lambda i: (i, 0)),
            pl.BlockSpec((ROW_BLK, D), lambda i: (i, 0)),
            pl.BlockSpec((ROW_BLK, 1), lambda i: (i, 0)),
            pl.BlockSpec((1, D), lambda i: (0, 0)),
        ],
        out_specs=pl.BlockSpec((ROW_BLK, D), lambda i: (i, 0)),
        out_shape=jax.ShapeDtypeStruct((N, D), jnp.float32),
    )(partial[0], partial[1], normc, bias.reshape(1, D))
    return out
